# baseline (speedup 1.0000x reference)
import functools

import jax
import jax.numpy as jnp
from jax import lax
from jax.experimental import pallas as pl
from jax.experimental.pallas import tpu as pltpu
from jax.experimental.pallas import tpu_sc as plsc

B = 16
S = 4096
LANES = 16
CHUNKS = S // LANES


def _make_sc_kernel():
  mesh = plsc.VectorSubcoreMesh(
      core_axis_name="c", subcore_axis_name="s", num_cores=1)

  @functools.partial(
      pl.kernel,
      mesh=mesh,
      out_type=jax.ShapeDtypeStruct((B, S), jnp.float32),
      scratch_types=[
          pltpu.VMEM((S,), jnp.float32),
          pltpu.SemaphoreType.DMA,
      ],
      compiler_params=pltpu.CompilerParams(needs_layout_passes=False),
  )
  def cumsum_kernel(pad_hbm, out_hbm, buf, sem):
    row = lax.axis_index("s")

    pltpu.async_copy(pad_hbm.at[row], buf, sem).wait()

    def scan_body(i, carry):
      base = i * LANES
      v = buf[pl.ds(base, LANES)]
      buf[pl.ds(base, LANES)] = plsc.cumsum(v) + carry
      return carry + jnp.sum(v)

    half = S // 2
    carry = lax.fori_loop(0, CHUNKS // 2, scan_body, jnp.float32(-1.0),
                          unroll=16)
    cp0 = pltpu.async_copy(buf.at[pl.ds(0, half)],
                           out_hbm.at[row, pl.ds(0, half)], sem)
    lax.fori_loop(CHUNKS // 2, CHUNKS, scan_body, carry, unroll=16)
    cp1 = pltpu.async_copy(buf.at[pl.ds(half, half)],
                           out_hbm.at[row, pl.ds(half, half)], sem)
    cp0.wait()
    cp1.wait()

  return cumsum_kernel


_sc_cumsum = _make_sc_kernel()


@jax.jit
def kernel(pad_masks):
  return _sc_cumsum(pad_masks)


# speedup vs baseline: 1.0221x; 1.0221x over previous
import functools

import jax
import jax.numpy as jnp
from jax import lax
from jax.experimental import pallas as pl
from jax.experimental.pallas import tpu as pltpu
from jax.experimental.pallas import tpu_sc as plsc

B = 16
S = 4096
LANES = 16
CHUNKS = S // LANES


def _make_sc_kernel():
  mesh = plsc.VectorSubcoreMesh(
      core_axis_name="c", subcore_axis_name="s", num_cores=1)

  @functools.partial(
      pl.kernel,
      mesh=mesh,
      out_type=jax.ShapeDtypeStruct((B, S), jnp.float32),
      scratch_types=[
          pltpu.VMEM((S,), jnp.float32),
          pltpu.SemaphoreType.DMA,
      ],
      compiler_params=pltpu.CompilerParams(needs_layout_passes=False),
  )
  def cumsum_kernel(pad_hbm, out_hbm, buf, sem):
    row = lax.axis_index("s")

    pltpu.async_copy(pad_hbm.at[row], buf, sem).wait()

    def scan_body(i, carry):
      base = i * LANES
      v = buf[pl.ds(base, LANES)]
      buf[pl.ds(base, LANES)] = plsc.cumsum(v) + carry
      return carry + jnp.sum(v)

    lax.fori_loop(0, CHUNKS, scan_body, jnp.float32(-1.0), unroll=16)

    pltpu.sync_copy(buf, out_hbm.at[row])

  return cumsum_kernel


_sc_cumsum = _make_sc_kernel()


@jax.jit
def kernel(pad_masks):
  return _sc_cumsum(pad_masks)
